# baseline (speedup 1.0000x reference)
import functools

import jax
import jax.numpy as jnp
from jax import lax
from jax.experimental import pallas as pl
from jax.experimental.pallas import tpu as pltpu
from jax.experimental.pallas import tpu_sc as plsc

N = 10000
D = 128
E = 320000
R = 5
G_S = 40000
S_ = 20000

NC = 2
NS = 16
NW = NC * NS
LANES = 16
NB = E // 128
BPW_MAX = NB // NW + 1
GBLK = 13
NGRP = (NB // NW) // GBLK
VPB = 128 // LANES
SUM_ROWS = NW * R + NW


def _sc_edge_filter(ei3_hbm, typ_hbm, x_hbm, sum_out,
                    dstv, acc, cntv, idxv, src16, typ16, rows, sem):
    wid = lax.axis_index("s") * NC + lax.axis_index("c")
    b0 = (NB * wid) // NW
    nblk = (NB * (wid + 1)) // NW - b0
    pltpu.sync_copy(ei3_hbm.at[pl.ds(b0, BPW_MAX), 1, :], dstv)

    zf = jnp.zeros((LANES,), jnp.float32)
    for r in range(R):
        for c in range(D // LANES):
            acc[r, pl.ds(c * LANES, LANES)] = zf
    cntv[...] = zf
    lanes_iota = lax.broadcasted_iota(jnp.int32, (LANES,), 0)

    def handle_vreg(blk, j):
        j16 = pl.multiple_of(j * LANES, LANES)
        d = dstv[blk, pl.ds(j16, LANES)]
        m = d == 0
        nh = jnp.sum(jnp.where(m, 1, 0))

        @pl.when(nh > 0)
        def _():
            pltpu.sync_copy(ei3_hbm.at[b0 + blk, 0, pl.ds(j16, LANES)], src16)
            goff = pl.multiple_of((b0 + blk) * 128 + j16, LANES)
            pltpu.sync_copy(typ_hbm.at[pl.ds(goff, LANES)], typ16)
            t = typ16[...]
            idxv[...] = jnp.where(m, src16[...], 0)
            pltpu.async_copy(x_hbm.at[idxv], rows, sem).wait()

            def lane_body(l, carry):
                sel = lanes_iota == l
                hit = jnp.sum(jnp.where(sel & m, 1, 0))

                @pl.when(hit > 0)
                def _():
                    t_l = jnp.sum(jnp.where(sel, t, 0))
                    for c in range(D // LANES):
                        acc[t_l, pl.ds(c * LANES, LANES)] += (
                            rows[l, pl.ds(c * LANES, LANES)])
                    cntv[...] += jnp.where(lanes_iota == t_l, 1.0, 0.0)

                return carry

            lax.fori_loop(0, LANES, lane_body, 0)

    def block_scan(blk):
        mn = dstv[blk, pl.ds(0, LANES)]
        for j in range(1, VPB):
            mn = jnp.minimum(mn, dstv[blk, pl.ds(j * LANES, LANES)])
        any0 = jnp.sum(jnp.where(mn == 0, 1, 0))

        @pl.when(any0 > 0)
        def _():
            def vbody(j, c2):
                handle_vreg(blk, j)
                return c2

            lax.fori_loop(0, VPB, vbody, 0)

    def group_body(g, carry):
        gb = g * GBLK
        mn = dstv[gb, pl.ds(0, LANES)]
        first = True
        for k in range(GBLK):
            for j in range(VPB):
                if first:
                    first = False
                    continue
                mn = jnp.minimum(mn, dstv[gb + k, pl.ds(j * LANES, LANES)])
        any0 = jnp.sum(jnp.where(mn == 0, 1, 0))

        @pl.when(any0 > 0)
        def _():
            def bbody(k, c2):
                block_scan(gb + k)
                return c2

            lax.fori_loop(0, GBLK, bbody, 0)

        return carry

    lax.fori_loop(0, NGRP, group_body, 0)

    @pl.when(nblk == BPW_MAX)
    def _():
        block_scan(NGRP * GBLK)

    pltpu.sync_copy(acc, sum_out.at[pl.ds(wid * R, R)])
    pltpu.sync_copy(cntv, sum_out.at[NW * R + wid, pl.ds(0, LANES)])


_sc_filter_call = functools.partial(
    pl.kernel,
    out_type=jax.ShapeDtypeStruct((SUM_ROWS, D), jnp.float32),
    mesh=plsc.VectorSubcoreMesh(core_axis_name="c", subcore_axis_name="s"),
    compiler_params=pltpu.CompilerParams(
        needs_layout_passes=False, use_tc_tiling_on_sc=False),
    scratch_types=[
        pltpu.VMEM((BPW_MAX, 128), jnp.int32),
        pltpu.VMEM((R, D), jnp.float32),
        pltpu.VMEM((LANES,), jnp.float32),
        pltpu.VMEM((LANES,), jnp.int32),
        pltpu.VMEM((LANES,), jnp.int32),
        pltpu.VMEM((LANES,), jnp.int32),
        pltpu.VMEM((LANES, D), jnp.float32),
        pltpu.SemaphoreType.DMA,
    ],
)(_sc_edge_filter)


def _tc_dense(x_ref, root_ref, bias_ref, basis_ref, comp_ref, sum_ref,
              wgt_ref, bg_ref, wst_ref, bs_ref, outg_ref, outs_ref,
              x0_scr, zg_scr, zs_scr, stats):
    i = pl.program_id(0)

    @pl.when(i == 0)
    def _():
        arr = sum_ref[...]
        sums = arr[:NW * R].reshape(NW, R, D)
        s = jnp.sum(sums, axis=0)
        cnt = jnp.sum(arr[NW * R:, :LANES], axis=0)
        den = jnp.maximum(cnt[:R], 1.0).reshape(R, 1)
        u = s / den
        v = jnp.einsum('rb,rd->bd', comp_ref[...], u)
        basis2 = basis_ref[...].reshape(R * D, D)
        msg = v.reshape(1, R * D) @ basis2
        out0 = x_ref[0:1, :] @ root_ref[...] + bias_ref[...] + msg
        x0_scr[...] = jnp.maximum(out0, 0.0)
        stats[0] = -jnp.inf
        stats[1] = 0.0
        stats[2] = -jnp.inf
        stats[3] = 0.0

    x0 = x0_scr[...]
    dn = (((1,), (1,)), ((), ()))

    col = lax.broadcasted_iota(jnp.int32, (1, CT), 1)

    @pl.when(i < NG_T)
    def _():
        zb = lax.dot_general(x0, wgt_ref[...], dn) + bg_ref[...]
        zb = jnp.where(col + i * CT < G_S, zb, -jnp.inf)
        k = pl.multiple_of(i * CT, CT)
        zg_scr[:, pl.ds(k, CT)] = zb
        m_old = stats[0]
        m_new = jnp.maximum(m_old, jnp.max(zb))
        stats[1] = stats[1] * jnp.exp(m_old - m_new) + jnp.sum(jnp.exp(zb - m_new))
        stats[0] = m_new

    @pl.when((i >= NG_T) & (i < NG_T + NS_T))
    def _():
        zb = lax.dot_general(x0, wst_ref[...], dn) + bs_ref[...]
        zb = jnp.where(col + (i - NG_T) * CT < S_, zb, -jnp.inf)
        k = pl.multiple_of((i - NG_T) * CT, CT)
        zs_scr[:, pl.ds(k, CT)] = zb
        m_old = stats[2]
        m_new = jnp.maximum(m_old, jnp.max(zb))
        stats[3] = stats[3] * jnp.exp(m_old - m_new) + jnp.sum(jnp.exp(zb - m_new))
        stats[2] = m_new

    @pl.when((i >= PHB) & (i < PHB + NG_T))
    def _():
        k = pl.multiple_of((i - PHB) * CT, CT)
        outg_ref[...] = (zg_scr[:, pl.ds(k, CT)]
                         - stats[0] - jnp.log(stats[1])).reshape(CT)

    @pl.when(i >= PHB + NG_T)
    def _():
        k = pl.multiple_of((i - PHB - NG_T) * CT, CT)
        outs_ref[...] = (zs_scr[:, pl.ds(k, CT)]
                         - stats[2] - jnp.log(stats[3])).reshape(CT)


def _full(shape):
    nd = len(shape)
    return pl.BlockSpec(shape, lambda i, n=nd: (0,) * n)


CT = 4096
NG_T = -(-G_S // CT)
NS_T = -(-S_ // CT)
PHB = NG_T + NS_T
GRID_T = 2 * PHB

_tc_dense_call = pl.pallas_call(
    _tc_dense,
    grid=(GRID_T,),
    in_specs=[
        pl.BlockSpec((8, D), lambda i: (0, 0)),
        _full((D, D)),
        _full((1, D)),
        _full((R, D, D)),
        _full((R, R)),
        _full((SUM_ROWS, D)),
        pl.BlockSpec((CT, D), lambda i: (jnp.clip(i, 0, NG_T - 1), 0)),
        pl.BlockSpec((1, CT), lambda i: (0, jnp.clip(i, 0, NG_T - 1))),
        pl.BlockSpec((CT, D), lambda i: (jnp.clip(i - NG_T, 0, NS_T - 1), 0)),
        pl.BlockSpec((1, CT), lambda i: (0, jnp.clip(i - NG_T, 0, NS_T - 1))),
    ],
    out_specs=[
        pl.BlockSpec((CT,), lambda i: (jnp.clip(i - PHB, 0, NG_T - 1),)),
        pl.BlockSpec((CT,), lambda i: (jnp.clip(i - PHB - NG_T, 0, NS_T - 1),)),
    ],
    out_shape=[
        jax.ShapeDtypeStruct((G_S,), jnp.float32),
        jax.ShapeDtypeStruct((S_,), jnp.float32),
    ],
    scratch_shapes=[
        pltpu.VMEM((1, D), jnp.float32),
        pltpu.VMEM((1, NG_T * CT), jnp.float32),
        pltpu.VMEM((1, NS_T * CT), jnp.float32),
        pltpu.SMEM((4,), jnp.float32),
    ],
)


@jax.jit
def kernel(x, edge_index, edge_type, basis, comp, root, conv_bias,
           Wg, bg, Ws, bs):
    ei3 = jnp.transpose(edge_index.reshape(2, NB, 128), (1, 0, 2))
    sums = _sc_filter_call(ei3, edge_type, x)
    outg, outs = _tc_dense_call(
        x, root, conv_bias.reshape(1, D), basis, comp, sums,
        Wg.T, bg.reshape(1, G_S), Ws.T, bs.reshape(1, S_))
    return outg, outs


# speedup vs baseline: 1.1800x; 1.1800x over previous
import functools

import jax
import jax.numpy as jnp
from jax import lax
from jax.experimental import pallas as pl
from jax.experimental.pallas import tpu as pltpu
from jax.experimental.pallas import tpu_sc as plsc

N = 10000
D = 128
E = 320000
R = 5
G_S = 40000
S_ = 20000

NC = 2
NS = 16
NW = NC * NS
LANES = 16
NB = E // 128
BPW_MAX = NB // NW + 1
GBLK = 13
NGRP = (NB // NW) // GBLK
VPB = 128 // LANES
SUM_ROWS = NW * R + NW


def _sc_edge_filter(ei3_hbm, typ_hbm, x_hbm, sum_out,
                    dstv, acc, cntv, idxv, src16, typ16, rows, sem, sem2):
    wid = lax.axis_index("s") * NC + lax.axis_index("c")
    b0 = (NB * wid) // NW
    nblk = (NB * (wid + 1)) // NW - b0
    pltpu.sync_copy(ei3_hbm.at[pl.ds(b0, BPW_MAX), 1, :], dstv)

    zf = jnp.zeros((LANES,), jnp.float32)
    for r in range(R):
        for c in range(D // LANES):
            acc[r, pl.ds(c * LANES, LANES)] = zf
    cntv[...] = zf
    lanes_iota = lax.broadcasted_iota(jnp.int32, (LANES,), 0)

    def handle_vreg(blk, j):
        j16 = pl.multiple_of(j * LANES, LANES)
        d = dstv[blk, pl.ds(j16, LANES)]
        m = d == 0
        nh = jnp.sum(jnp.where(m, 1, 0))

        @pl.when(nh > 0)
        def _():
            c1 = pltpu.async_copy(
                ei3_hbm.at[b0 + blk, 0, pl.ds(j16, LANES)], src16, sem)
            goff = pl.multiple_of((b0 + blk) * 128 + j16, LANES)
            c2 = pltpu.async_copy(typ_hbm.at[pl.ds(goff, LANES)], typ16, sem2)
            c1.wait()
            idxv[...] = jnp.where(m, src16[...], 0)
            cg = pltpu.async_copy(x_hbm.at[idxv], rows, sem)
            c2.wait()
            t = typ16[...]
            cg.wait()

            def lane_body(l, carry):
                sel = lanes_iota == l
                hit = jnp.sum(jnp.where(sel & m, 1, 0))

                @pl.when(hit > 0)
                def _():
                    t_l = jnp.sum(jnp.where(sel, t, 0))
                    for c in range(D // LANES):
                        acc[t_l, pl.ds(c * LANES, LANES)] += (
                            rows[l, pl.ds(c * LANES, LANES)])
                    cntv[...] += jnp.where(lanes_iota == t_l, 1.0, 0.0)

                return carry

            lax.fori_loop(0, LANES, lane_body, 0)

    def block_scan(blk):
        mn = dstv[blk, pl.ds(0, LANES)]
        for j in range(1, VPB):
            mn = jnp.minimum(mn, dstv[blk, pl.ds(j * LANES, LANES)])
        any0 = jnp.sum(jnp.where(mn == 0, 1, 0))

        @pl.when(any0 > 0)
        def _():
            def vbody(j, c2):
                handle_vreg(blk, j)
                return c2

            lax.fori_loop(0, VPB, vbody, 0)

    def group_body(g, carry):
        gb = g * GBLK
        mn = dstv[gb, pl.ds(0, LANES)]
        first = True
        for k in range(GBLK):
            for j in range(VPB):
                if first:
                    first = False
                    continue
                mn = jnp.minimum(mn, dstv[gb + k, pl.ds(j * LANES, LANES)])
        any0 = jnp.sum(jnp.where(mn == 0, 1, 0))

        @pl.when(any0 > 0)
        def _():
            def bbody(k, c2):
                block_scan(gb + k)
                return c2

            lax.fori_loop(0, GBLK, bbody, 0)

        return carry

    lax.fori_loop(0, NGRP, group_body, 0)

    @pl.when(nblk == BPW_MAX)
    def _():
        block_scan(NGRP * GBLK)

    pltpu.sync_copy(acc, sum_out.at[pl.ds(wid * R, R)])
    pltpu.sync_copy(cntv, sum_out.at[NW * R + wid, pl.ds(0, LANES)])


_sc_filter_call = functools.partial(
    pl.kernel,
    out_type=jax.ShapeDtypeStruct((SUM_ROWS, D), jnp.float32),
    mesh=plsc.VectorSubcoreMesh(core_axis_name="c", subcore_axis_name="s"),
    compiler_params=pltpu.CompilerParams(
        needs_layout_passes=False, use_tc_tiling_on_sc=False),
    scratch_types=[
        pltpu.VMEM((BPW_MAX, 128), jnp.int32),
        pltpu.VMEM((R, D), jnp.float32),
        pltpu.VMEM((LANES,), jnp.float32),
        pltpu.VMEM((LANES,), jnp.int32),
        pltpu.VMEM((LANES,), jnp.int32),
        pltpu.VMEM((LANES,), jnp.int32),
        pltpu.VMEM((LANES, D), jnp.float32),
        pltpu.SemaphoreType.DMA,
        pltpu.SemaphoreType.DMA,
    ],
)(_sc_edge_filter)


def _tc_dense(x_ref, root_ref, bias_ref, basis_ref, comp_ref, sum_ref,
              wgt_ref, bg_ref, wst_ref, bs_ref, outg_ref, outs_ref):
    arr = sum_ref[...]
    sums = arr[:NW * R].reshape(NW, R, D)
    s = jnp.sum(sums, axis=0)
    cnt = jnp.sum(arr[NW * R:, :LANES], axis=0)
    den = jnp.maximum(cnt[:R], 1.0).reshape(R, 1)
    u = s / den
    v = jnp.einsum('rb,rd->bd', comp_ref[...], u)
    basis2 = basis_ref[...].reshape(R * D, D)
    msg = v.reshape(1, R * D) @ basis2
    out0 = x_ref[0:1, :] @ root_ref[...] + bias_ref[...] + msg
    x0 = jnp.maximum(out0, 0.0)

    dn = (((1,), (1,)), ((), ()))
    zg = lax.dot_general(x0, wgt_ref[...], dn) + bg_ref[...]
    mg = jnp.max(zg)
    lg = jnp.log(jnp.sum(jnp.exp(zg - mg)))
    outg_ref[...] = (zg - mg - lg).reshape(G_S)

    zs = lax.dot_general(x0, wst_ref[...], dn) + bs_ref[...]
    ms = jnp.max(zs)
    ls = jnp.log(jnp.sum(jnp.exp(zs - ms)))
    outs_ref[...] = (zs - ms - ls).reshape(S_)


def _full(shape):
    nd = len(shape)
    return pl.BlockSpec(shape, lambda i, n=nd: (0,) * n)


_tc_dense_call = pl.pallas_call(
    _tc_dense,
    grid=(1,),
    in_specs=[
        pl.BlockSpec((8, D), lambda i: (0, 0)),
        _full((D, D)),
        _full((1, D)),
        _full((R, D, D)),
        _full((R, R)),
        _full((SUM_ROWS, D)),
        _full((G_S, D)),
        _full((1, G_S)),
        _full((S_, D)),
        _full((1, S_)),
    ],
    out_specs=[
        pl.BlockSpec((G_S,), lambda i: (0,)),
        pl.BlockSpec((S_,), lambda i: (0,)),
    ],
    out_shape=[
        jax.ShapeDtypeStruct((G_S,), jnp.float32),
        jax.ShapeDtypeStruct((S_,), jnp.float32),
    ],
)


@jax.jit
def kernel(x, edge_index, edge_type, basis, comp, root, conv_bias,
           Wg, bg, Ws, bs):
    ei3 = jnp.transpose(edge_index.reshape(2, NB, 128), (1, 0, 2))
    sums = _sc_filter_call(ei3, edge_type, x)
    outg, outs = _tc_dense_call(
        x, root, conv_bias.reshape(1, D), basis, comp, sums,
        Wg.T, bg.reshape(1, G_S), Ws.T, bs.reshape(1, S_))
    return outg, outs


# speedup vs baseline: 1.1886x; 1.0073x over previous
import functools

import jax
import jax.numpy as jnp
from jax import lax
from jax.experimental import pallas as pl
from jax.experimental.pallas import tpu as pltpu
from jax.experimental.pallas import tpu_sc as plsc

N = 10000
D = 128
E = 320000
R = 5
G_S = 40000
S_ = 20000

NC = 2
NS = 16
NW = NC * NS
LANES = 16
NB = E // 128
BPW_MAX = NB // NW + 1
GBLK = 13
NGRP = (NB // NW) // GBLK
VPB = 128 // LANES
SUM_ROWS = NW * R + NW


def _sc_edge_filter(ei3_hbm, typ_hbm, x_hbm, sum_out,
                    dstv, acc, cntv, hits, idx2, src2, typ2, rows2,
                    nctr, sem_g, sem_f):
    wid = lax.axis_index("s") * NC + lax.axis_index("c")
    b0 = (NB * wid) // NW
    nblk = (NB * (wid + 1)) // NW - b0
    pltpu.sync_copy(ei3_hbm.at[pl.ds(b0, BPW_MAX), 1, :], dstv)

    zf = jnp.zeros((LANES,), jnp.float32)
    for r in range(R):
        for c in range(D // LANES):
            acc[r, pl.ds(c * LANES, LANES)] = zf
    cntv[...] = zf
    nctr[0] = 0
    lanes_iota = lax.broadcasted_iota(jnp.int32, (LANES,), 0)
    zi = jnp.zeros((LANES,), jnp.int32)

    def record_vreg(blk, j):
        j16 = pl.multiple_of(j * LANES, LANES)
        d = dstv[blk, pl.ds(j16, LANES)]
        nh = jnp.sum(jnp.where(d == 0, 1, 0))

        @pl.when(nh > 0)
        def _():
            n = nctr[0]
            hits[n, :] = zi + blk * VPB + j
            nctr[0] = n + 1

    def block_scan(blk):
        mn = dstv[blk, pl.ds(0, LANES)]
        for j in range(1, VPB):
            mn = jnp.minimum(mn, dstv[blk, pl.ds(j * LANES, LANES)])
        any0 = jnp.sum(jnp.where(mn == 0, 1, 0))

        @pl.when(any0 > 0)
        def _():
            def vbody(j, c2):
                record_vreg(blk, j)
                return c2

            lax.fori_loop(0, VPB, vbody, 0)

    def group_body(g, carry):
        gb = g * GBLK
        mn = dstv[gb, pl.ds(0, LANES)]
        first = True
        for k in range(GBLK):
            for j in range(VPB):
                if first:
                    first = False
                    continue
                mn = jnp.minimum(mn, dstv[gb + k, pl.ds(j * LANES, LANES)])
        any0 = jnp.sum(jnp.where(mn == 0, 1, 0))

        @pl.when(any0 > 0)
        def _():
            def bbody(k, c2):
                block_scan(gb + k)
                return c2

            lax.fori_loop(0, GBLK, bbody, 0)

        return carry

    lax.fori_loop(0, NGRP, group_body, 0)

    @pl.when(nblk == BPW_MAX)
    def _():
        block_scan(NGRP * GBLK)

    ntot = nctr[0]

    def hit_code(i):
        return jnp.sum(jnp.where(lanes_iota == 0, hits[i, :], 0))

    def fetch(i, slot):
        code = hit_code(i)
        blk = code // VPB
        j16 = pl.multiple_of((code % VPB) * LANES, LANES)
        pltpu.async_copy(
            ei3_hbm.at[b0 + blk, 0, pl.ds(j16, LANES)], src2.at[slot], sem_f)
        goff = pl.multiple_of((b0 + blk) * 128 + j16, LANES)
        pltpu.async_copy(typ_hbm.at[pl.ds(goff, LANES)], typ2.at[slot], sem_f)

    @pl.when(ntot > 0)
    def _():
        fetch(0, 0)

        def loop_body(i, carry):
            slot = lax.rem(i, 2)
            code = hit_code(i)
            blk = code // VPB
            j16 = pl.multiple_of((code % VPB) * LANES, LANES)
            m = dstv[blk, pl.ds(j16, LANES)] == 0
            pltpu.make_async_copy(
                ei3_hbm.at[0, 0, pl.ds(0, LANES)], src2.at[slot],
                sem_f).wait()
            pltpu.make_async_copy(
                typ_hbm.at[pl.ds(0, LANES)], typ2.at[slot], sem_f).wait()
            idx2[slot, :] = jnp.where(m, src2[slot, :], 0)
            h = pltpu.async_copy(x_hbm.at[idx2.at[slot]], rows2.at[slot],
                                 sem_g)

            @pl.when(i + 1 < ntot)
            def _():
                fetch(i + 1, 1 - slot)

            h.wait()
            t = typ2[slot, :]

            def lane_cond(mvi):
                return jnp.sum(mvi) > 0

            def lane_step(mvi):
                lsp = plsc.all_reduce_ffs(mvi != 0)
                l = jnp.sum(jnp.where(lanes_iota == 0, lsp, 0))
                t_l = jnp.sum(jnp.where(lanes_iota == l, t, 0))
                for c in range(D // LANES):
                    acc[t_l, pl.ds(c * LANES, LANES)] += (
                        rows2[slot, l, pl.ds(c * LANES, LANES)])
                cntv[...] += jnp.where(lanes_iota == t_l, 1.0, 0.0)
                return jnp.where(lanes_iota == l, 0, mvi)

            lax.while_loop(lane_cond, lane_step, jnp.where(m, 1, 0))
            return carry

        lax.fori_loop(0, ntot, loop_body, 0)

    pltpu.sync_copy(acc, sum_out.at[pl.ds(wid * R, R)])
    pltpu.sync_copy(cntv, sum_out.at[NW * R + wid, pl.ds(0, LANES)])


_sc_filter_call = functools.partial(
    pl.kernel,
    out_type=jax.ShapeDtypeStruct((SUM_ROWS, D), jnp.float32),
    mesh=plsc.VectorSubcoreMesh(core_axis_name="c", subcore_axis_name="s"),
    compiler_params=pltpu.CompilerParams(
        needs_layout_passes=False, use_tc_tiling_on_sc=False),
    scratch_types=[
        pltpu.VMEM((BPW_MAX, 128), jnp.int32),
        pltpu.VMEM((R, D), jnp.float32),
        pltpu.VMEM((LANES,), jnp.float32),
        pltpu.VMEM((BPW_MAX * VPB, LANES), jnp.int32),
        pltpu.VMEM((2, LANES), jnp.int32),
        pltpu.VMEM((2, LANES), jnp.int32),
        pltpu.VMEM((2, LANES), jnp.int32),
        pltpu.VMEM((2, LANES, D), jnp.float32),
        pltpu.SMEM((1,), jnp.int32),
        pltpu.SemaphoreType.DMA,
        pltpu.SemaphoreType.DMA,
    ],
)(_sc_edge_filter)


def _tc_dense(x_ref, root_ref, bias_ref, basis_ref, comp_ref, sum_ref,
              wgt_ref, bg_ref, wst_ref, bs_ref, outg_ref, outs_ref):
    arr = sum_ref[...]
    sums = arr[:NW * R].reshape(NW, R, D)
    s = jnp.sum(sums, axis=0)
    cnt = jnp.sum(arr[NW * R:, :LANES], axis=0)
    den = jnp.maximum(cnt[:R], 1.0).reshape(R, 1)
    u = s / den
    v = jnp.einsum('rb,rd->bd', comp_ref[...], u)
    basis2 = basis_ref[...].reshape(R * D, D)
    msg = v.reshape(1, R * D) @ basis2
    out0 = x_ref[0:1, :] @ root_ref[...] + bias_ref[...] + msg
    x0 = jnp.maximum(out0, 0.0)

    dn = (((1,), (1,)), ((), ()))
    zg = lax.dot_general(x0, wgt_ref[...], dn) + bg_ref[...]
    mg = jnp.max(zg)
    lg = jnp.log(jnp.sum(jnp.exp(zg - mg)))
    outg_ref[...] = (zg - mg - lg).reshape(G_S)

    zs = lax.dot_general(x0, wst_ref[...], dn) + bs_ref[...]
    ms = jnp.max(zs)
    ls = jnp.log(jnp.sum(jnp.exp(zs - ms)))
    outs_ref[...] = (zs - ms - ls).reshape(S_)


def _full(shape):
    nd = len(shape)
    return pl.BlockSpec(shape, lambda i, n=nd: (0,) * n)


_tc_dense_call = pl.pallas_call(
    _tc_dense,
    grid=(1,),
    in_specs=[
        pl.BlockSpec((8, D), lambda i: (0, 0)),
        _full((D, D)),
        _full((1, D)),
        _full((R, D, D)),
        _full((R, R)),
        _full((SUM_ROWS, D)),
        _full((G_S, D)),
        _full((1, G_S)),
        _full((S_, D)),
        _full((1, S_)),
    ],
    out_specs=[
        pl.BlockSpec((G_S,), lambda i: (0,)),
        pl.BlockSpec((S_,), lambda i: (0,)),
    ],
    out_shape=[
        jax.ShapeDtypeStruct((G_S,), jnp.float32),
        jax.ShapeDtypeStruct((S_,), jnp.float32),
    ],
)


@jax.jit
def kernel(x, edge_index, edge_type, basis, comp, root, conv_bias,
           Wg, bg, Ws, bs):
    ei3 = jnp.transpose(edge_index.reshape(2, NB, 128), (1, 0, 2))
    sums = _sc_filter_call(ei3, edge_type, x)
    outg, outs = _tc_dense_call(
        x, root, conv_bias.reshape(1, D), basis, comp, sums,
        Wg.T, bg.reshape(1, G_S), Ws.T, bs.reshape(1, S_))
    return outg, outs
